# baseline (speedup 1.0000x reference)
import functools

import jax
import jax.numpy as jnp
from jax import lax
from jax.experimental import pallas as pl
from jax.experimental.pallas import tpu as pltpu
from jax.experimental.pallas import tpu_sc as plsc

D = 64
L = 200
LANES = 16
DV = D // LANES

NC, NS = 2, 16
NW = NC * NS

TOTAL_ROWS = 4096 * 200
ROWS_PER_W = TOTAL_ROWS // NW
SEQ_PER_CHUNK = 4
CHUNK_ROWS = SEQ_PER_CHUNK * L
N_CHUNKS = ROWS_PER_W // CHUNK_ROWS
G = 80
N_G = CHUNK_ROWS // G


def _emb_body(seq_hbm, pos_hbm, tok_hbm, out_hbm, idx_v, rows_v, pos_v, sem):
    wid = lax.axis_index("s") * NC + lax.axis_index("c")
    base = wid * ROWS_PER_W

    pltpu.sync_copy(pos_hbm, pos_v)

    def chunk_body(ci, carry):
        cbase = base + ci * CHUNK_ROWS
        pltpu.sync_copy(seq_hbm.at[pl.ds(cbase, CHUNK_ROWS)], idx_v)
        copies = [
            pltpu.async_copy(
                tok_hbm.at[idx_v.at[pl.ds(j * G, G)]],
                rows_v.at[pl.ds(j * G, G)],
                sem,
            )
            for j in range(N_G)
        ]
        for c in copies:
            c.wait()

        def l_body(l, carry2):
            pv = [pos_v[l * DV + c] for c in range(DV)]
            for s in range(SEQ_PER_CHUNK):
                r = s * L + l
                for c in range(DV):
                    sl = pl.ds(c * LANES, LANES)
                    rows_v[r, sl] = rows_v[r, sl] + pv[c]
            return carry2

        lax.fori_loop(0, L, l_body, 0)
        pltpu.sync_copy(rows_v, out_hbm.at[pl.ds(cbase, CHUNK_ROWS)])
        return carry

    lax.fori_loop(0, N_CHUNKS, chunk_body, 0)


_emb = functools.partial(
    pl.kernel,
    out_type=jax.ShapeDtypeStruct((TOTAL_ROWS, D), jnp.float32),
    mesh=plsc.VectorSubcoreMesh(core_axis_name="c", subcore_axis_name="s"),
    scratch_types=[
        pltpu.VMEM((CHUNK_ROWS,), jnp.int32),
        pltpu.VMEM((CHUNK_ROWS, D), jnp.float32),
        pltpu.VMEM((L * DV, LANES), jnp.float32),
        pltpu.SemaphoreType.DMA,
    ],
    compiler_params=pltpu.CompilerParams(use_tc_tiling_on_sc=False),
)(_emb_body)


@jax.jit
def kernel(seq, token_table, pos_table):
    B, Lx = seq.shape
    seq_flat = seq.reshape(B * Lx).astype(jnp.int32)
    pos_flat = pos_table.reshape(L * DV, LANES)
    out = _emb(seq_flat, pos_flat, token_table)
    return out.reshape(B, Lx, D)
